# baseline (speedup 1.0000x reference)
import functools

import jax
import jax.numpy as jnp
from jax import lax
from jax.experimental import pallas as pl
from jax.experimental.pallas import tpu as pltpu
from jax.experimental.pallas import tpu_sc as plsc

E = 16
D = 2048
F = 4096
N = 2048
LW = 128
BM = 64
NT = 80
NP = NT * BM
BN1 = 1024
BN2 = 512
W = 16


def _gate_route_body(h_ref, gwt_ref, pos_ref, gw_ref, eid_ref, nt_ref,
                     o1_ref, o2_ref, r1_ref, r2_ref):
    f32 = jnp.float32
    h = h_ref[...].astype(jnp.bfloat16)
    logits = jnp.dot(h, gwt_ref[...].astype(jnp.bfloat16),
                     preferred_element_type=f32)
    lane = lax.broadcasted_iota(jnp.int32, (N, LW), 1)
    valid = lane < E
    logits = jnp.where(valid, logits, -1e30)
    m = jnp.max(logits, axis=1, keepdims=True)
    p = jnp.where(valid, jnp.exp(logits - m), 0.0)
    scores = p / jnp.sum(p, axis=1, keepdims=True)
    m1 = jnp.max(scores, axis=1, keepdims=True)
    i1 = jnp.min(jnp.where(scores == m1, lane, E), axis=1, keepdims=True)
    oh1 = lane == i1
    sc2 = jnp.where(oh1, -1.0, scores)
    m2 = jnp.max(sc2, axis=1, keepdims=True)
    i2 = jnp.min(jnp.where(sc2 == m2, lane, E), axis=1, keepdims=True)
    oh2 = lane == i2
    gw_ref[:, 0:1] = m1
    gw_ref[:, 1:2] = m2
    o1_ref[...] = oh1.astype(f32)
    o2_ref[...] = oh2.astype(f32)
    c1 = jnp.sum(o1_ref[...], axis=0, keepdims=True)
    c2 = jnp.sum(o2_ref[...], axis=0, keepdims=True)
    counts = c1 + c2
    tilecnt = jnp.floor((counts + (BM - 1)) * (1.0 / BM))
    row = lax.broadcasted_iota(jnp.int32, (LW, LW), 0)
    col = lax.broadcasted_iota(jnp.int32, (LW, LW), 1)
    upper = (row < col).astype(f32)
    poff_t = jnp.dot(tilecnt, upper, preferred_element_type=f32)
    poff_r = poff_t * BM
    nt_ref[...] = jnp.sum(tilecnt).astype(jnp.int32).reshape(1, 1)
    strict = (col < row).astype(f32)
    nb = N // LW

    def blk(b, carry, o_ref, r_ref):
        x = o_ref[pl.ds(b * LW, LW), :]
        cum = jnp.dot(strict, x, preferred_element_type=f32) + carry
        r_ref[pl.ds(b * LW, LW), :] = jnp.sum(x * cum, axis=1, keepdims=True)
        return carry + jnp.sum(x, axis=0, keepdims=True)

    carry = lax.fori_loop(0, nb, lambda b, c: blk(b, c, o1_ref, r1_ref),
                          jnp.zeros((1, LW), f32))
    lax.fori_loop(0, nb, lambda b, c: blk(b, c, o2_ref, r2_ref), carry)
    base1 = jnp.sum(o1_ref[...] * poff_r, axis=1, keepdims=True)
    base2 = jnp.sum(o2_ref[...] * poff_r, axis=1, keepdims=True)
    pos_ref[:, 0:1] = (base1 + r1_ref[...]).astype(jnp.int32)
    pos_ref[:, 1:2] = (base2 + r2_ref[...]).astype(jnp.int32)
    trow = lax.broadcasted_iota(jnp.int32, (LW, LW), 0).astype(f32)
    started = jnp.where((col < E) & (poff_t <= trow), 1.0, 0.0)
    eid_ref[...] = (jnp.sum(started, axis=1, keepdims=True) - 1.0).astype(jnp.int32)


def _gate_route(h, gwt):
    f32 = jnp.float32
    return pl.pallas_call(
        _gate_route_body,
        out_shape=(
            jax.ShapeDtypeStruct((N, 2), jnp.int32),
            jax.ShapeDtypeStruct((N, 2), f32),
            jax.ShapeDtypeStruct((LW, 1), jnp.int32),
            jax.ShapeDtypeStruct((1, 1), jnp.int32),
        ),
        scratch_shapes=[
            pltpu.VMEM((N, LW), f32), pltpu.VMEM((N, LW), f32),
            pltpu.VMEM((N, 1), f32), pltpu.VMEM((N, 1), f32),
        ],
    )(h, gwt)


def _dispatch(h, p0, p1):
    mesh = plsc.VectorSubcoreMesh(core_axis_name="core",
                                  subcore_axis_name="subcore")

    @functools.partial(
        pl.kernel,
        out_type=jax.ShapeDtypeStruct((NP, D), jnp.float32),
        mesh=mesh)
    def k(h_hbm, p0_hbm, p1_hbm, xp_hbm):
        def body(x_vmem, i0_vmem, i1_vmem):
            pltpu.sync_copy(x_vmem, xp_hbm.at[i0_vmem.at[0]])
            pltpu.sync_copy(x_vmem, xp_hbm.at[i1_vmem.at[0]])

        pltpu.emit_pipeline(
            body,
            grid=(N // W,),
            in_specs=[
                pl.BlockSpec((W, D), lambda i: (i, 0)),
                pl.BlockSpec((1, W), lambda i: (i, 0)),
                pl.BlockSpec((1, W), lambda i: (i, 0)),
            ],
            out_specs=[],
            core_axis_name=("core", "subcore"),
            dimension_semantics=(pltpu.PARALLEL,),
        )(h_hbm, p0_hbm, p1_hbm)

    return k(h, p0, p1)


def _ffn1_body(eid_ref, nt_ref, x_ref, w_ref, a_ref):
    t = pl.program_id(1)

    @pl.when(t < nt_ref[0])
    def _():
        x = x_ref[pl.ds(t * BM, BM), :].astype(jnp.bfloat16)
        w = w_ref[0].astype(jnp.bfloat16)
        y = jnp.dot(x, w, preferred_element_type=jnp.float32)
        a_ref[...] = (y / (1.0 + jnp.exp(-y))).astype(jnp.bfloat16)


def _ffn1(eid, nt, xp, w1):
    grid_spec = pltpu.PrefetchScalarGridSpec(
        num_scalar_prefetch=2,
        grid=(F // BN1, NT),
        in_specs=[
            pl.BlockSpec((NP, D), lambda n, t, eid, nt: (0, 0)),
            pl.BlockSpec((1, D, BN1), lambda n, t, eid, nt: (eid[t], 0, n)),
        ],
        out_specs=pl.BlockSpec((BM, BN1), lambda n, t, eid, nt: (t, n)),
    )
    return pl.pallas_call(
        _ffn1_body,
        grid_spec=grid_spec,
        out_shape=jax.ShapeDtypeStruct((NP, F), jnp.bfloat16),
    )(eid, nt, xp, w1)


def _ffn2_body(eid_ref, nt_ref, a_ref, w_ref, y_ref):
    t = pl.program_id(1)

    @pl.when(t < nt_ref[0])
    def _():
        a = a_ref[pl.ds(t * BM, BM), :]
        w = w_ref[0].astype(jnp.bfloat16)
        y_ref[...] = jnp.dot(a, w, preferred_element_type=jnp.float32)


def _ffn2(eid, nt, a, w2):
    grid_spec = pltpu.PrefetchScalarGridSpec(
        num_scalar_prefetch=2,
        grid=(D // BN2, NT),
        in_specs=[
            pl.BlockSpec((NP, F), lambda n, t, eid, nt: (0, 0)),
            pl.BlockSpec((1, F, BN2), lambda n, t, eid, nt: (eid[t], 0, n)),
        ],
        out_specs=pl.BlockSpec((BM, BN2), lambda n, t, eid, nt: (t, n)),
    )
    return pl.pallas_call(
        _ffn2_body,
        grid_spec=grid_spec,
        out_shape=jax.ShapeDtypeStruct((NP, D), jnp.float32),
    )(eid, nt, a, w2)


def _gather(yp, pos_flat):
    mesh = plsc.VectorSubcoreMesh(core_axis_name="core",
                                  subcore_axis_name="subcore")

    @functools.partial(
        pl.kernel,
        out_type=jax.ShapeDtypeStruct((2 * N, D), jnp.float32),
        mesh=mesh)
    def k(y_hbm, p_hbm, o_hbm):
        def body(i_vmem, o_vmem):
            pltpu.sync_copy(y_hbm.at[i_vmem.at[0]], o_vmem)

        pltpu.emit_pipeline(
            body,
            grid=(2 * N // W,),
            in_specs=[pl.BlockSpec((1, W), lambda i: (i, 0))],
            out_specs=[pl.BlockSpec((W, D), lambda i: (i, 0))],
            core_axis_name=("core", "subcore"),
            dimension_semantics=(pltpu.PARALLEL,),
        )(p_hbm, o_hbm)

    return k(yp, pos_flat)


def _combine_body(y0_ref, y1_ref, g0_ref, g1_ref, o_ref):
    o_ref[...] = y0_ref[...] * g0_ref[...] + y1_ref[...] * g1_ref[...]


def _combine(yg, g0, g1):
    nb = N // LW
    return pl.pallas_call(
        _combine_body,
        grid=(nb,),
        in_specs=[
            pl.BlockSpec((LW, D), lambda i: (i, 0)),
            pl.BlockSpec((LW, D), lambda i: (i + nb, 0)),
            pl.BlockSpec((LW, 1), lambda i: (i, 0)),
            pl.BlockSpec((LW, 1), lambda i: (i, 0)),
        ],
        out_specs=pl.BlockSpec((LW, D), lambda i: (i, 0)),
        out_shape=jax.ShapeDtypeStruct((N, D), jnp.float32),
    )(yg, yg, g0, g1)


def kernel(h, gate_weight, w1, w2):
    gwt = jnp.zeros((D, LW), jnp.float32).at[:, :E].set(gate_weight.T)
    pos, gw, eid_col, nt = _gate_route(h, gwt)
    pos_rows = pos.T
    eid = eid_col.reshape(LW)[:NT]
    nt = nt.reshape(1)
    xp = _dispatch(h, pos_rows[0].reshape(N // W, W),
                   pos_rows[1].reshape(N // W, W))
    a = _ffn1(eid, nt, xp, w1)
    yp = _ffn2(eid, nt, a, w2)
    yg = _gather(yp, pos_rows.reshape(2 * N // W, W))
    return _combine(yg, gw[:, 0:1], gw[:, 1:2])


# speedup vs baseline: 1.1360x; 1.1360x over previous
import functools

import jax
import jax.numpy as jnp
from jax import lax
from jax.experimental import pallas as pl
from jax.experimental.pallas import tpu as pltpu
from jax.experimental.pallas import tpu_sc as plsc

E = 16
D = 2048
F = 4096
N = 2048
LW = 128
BM = 64
NT = 80
NP = NT * BM
BN1 = 2048
BN2 = 1024
W = 16


def _gate_route_body(h_ref, gwt_ref, pos_ref, gw_ref, eid_ref, nt_ref,
                     o1_ref, o2_ref, r1_ref, r2_ref):
    f32 = jnp.float32
    h = h_ref[...].astype(jnp.bfloat16)
    logits = jnp.dot(h, gwt_ref[...].astype(jnp.bfloat16),
                     preferred_element_type=f32)
    lane = lax.broadcasted_iota(jnp.int32, (N, LW), 1)
    valid = lane < E
    logits = jnp.where(valid, logits, -1e30)
    m = jnp.max(logits, axis=1, keepdims=True)
    p = jnp.where(valid, jnp.exp(logits - m), 0.0)
    scores = p / jnp.sum(p, axis=1, keepdims=True)
    m1 = jnp.max(scores, axis=1, keepdims=True)
    i1 = jnp.min(jnp.where(scores == m1, lane, E), axis=1, keepdims=True)
    oh1 = lane == i1
    sc2 = jnp.where(oh1, -1.0, scores)
    m2 = jnp.max(sc2, axis=1, keepdims=True)
    i2 = jnp.min(jnp.where(sc2 == m2, lane, E), axis=1, keepdims=True)
    oh2 = lane == i2
    gw_ref[:, 0:1] = m1
    gw_ref[:, 1:2] = m2
    o1_ref[...] = oh1.astype(f32)
    o2_ref[...] = oh2.astype(f32)
    c1 = jnp.sum(o1_ref[...], axis=0, keepdims=True)
    c2 = jnp.sum(o2_ref[...], axis=0, keepdims=True)
    counts = c1 + c2
    tilecnt = jnp.floor((counts + (BM - 1)) * (1.0 / BM))
    row = lax.broadcasted_iota(jnp.int32, (LW, LW), 0)
    col = lax.broadcasted_iota(jnp.int32, (LW, LW), 1)
    upper = (row < col).astype(f32)
    poff_t = jnp.dot(tilecnt, upper, preferred_element_type=f32)
    poff_r = poff_t * BM
    nt_ref[...] = jnp.sum(tilecnt).astype(jnp.int32).reshape(1, 1)
    strict = (col < row).astype(f32)
    nb = N // LW

    def blk(b, carry, o_ref, r_ref):
        x = o_ref[pl.ds(b * LW, LW), :]
        cum = jnp.dot(strict, x, preferred_element_type=f32) + carry
        r_ref[pl.ds(b * LW, LW), :] = jnp.sum(x * cum, axis=1, keepdims=True)
        return carry + jnp.sum(x, axis=0, keepdims=True)

    carry = lax.fori_loop(0, nb, lambda b, c: blk(b, c, o1_ref, r1_ref),
                          jnp.zeros((1, LW), f32))
    lax.fori_loop(0, nb, lambda b, c: blk(b, c, o2_ref, r2_ref), carry)
    base1 = jnp.sum(o1_ref[...] * poff_r, axis=1, keepdims=True)
    base2 = jnp.sum(o2_ref[...] * poff_r, axis=1, keepdims=True)
    pos_ref[:, 0:1] = (base1 + r1_ref[...]).astype(jnp.int32)
    pos_ref[:, 1:2] = (base2 + r2_ref[...]).astype(jnp.int32)
    trow = lax.broadcasted_iota(jnp.int32, (LW, LW), 0).astype(f32)
    started = jnp.where((col < E) & (poff_t <= trow), 1.0, 0.0)
    eid_ref[...] = (jnp.sum(started, axis=1, keepdims=True) - 1.0).astype(jnp.int32)


def _gate_route(h, gwt):
    f32 = jnp.float32
    return pl.pallas_call(
        _gate_route_body,
        out_shape=(
            jax.ShapeDtypeStruct((N, 2), jnp.int32),
            jax.ShapeDtypeStruct((N, 2), f32),
            jax.ShapeDtypeStruct((LW, 1), jnp.int32),
            jax.ShapeDtypeStruct((1, 1), jnp.int32),
        ),
        scratch_shapes=[
            pltpu.VMEM((N, LW), f32), pltpu.VMEM((N, LW), f32),
            pltpu.VMEM((N, 1), f32), pltpu.VMEM((N, 1), f32),
        ],
    )(h, gwt)


def _dispatch(h, p0, p1):
    mesh = plsc.VectorSubcoreMesh(core_axis_name="core",
                                  subcore_axis_name="subcore")

    @functools.partial(
        pl.kernel,
        out_type=jax.ShapeDtypeStruct((NP, D), jnp.float32),
        mesh=mesh)
    def k(h_hbm, p0_hbm, p1_hbm, xp_hbm):
        def body(x_vmem, i0_vmem, i1_vmem):
            pltpu.sync_copy(x_vmem, xp_hbm.at[i0_vmem.at[0]])
            pltpu.sync_copy(x_vmem, xp_hbm.at[i1_vmem.at[0]])

        pltpu.emit_pipeline(
            body,
            grid=(N // W,),
            in_specs=[
                pl.BlockSpec((W, D), lambda i: (i, 0)),
                pl.BlockSpec((1, W), lambda i: (i, 0)),
                pl.BlockSpec((1, W), lambda i: (i, 0)),
            ],
            out_specs=[],
            core_axis_name=("core", "subcore"),
            dimension_semantics=(pltpu.PARALLEL,),
        )(h_hbm, p0_hbm, p1_hbm)

    return k(h, p0, p1)


def _ffn1_body(eid_ref, nt_ref, x_ref, w_ref, a_ref):
    t = pl.program_id(1)

    @pl.when(t < nt_ref[0])
    def _():
        x = x_ref[...].astype(jnp.bfloat16)
        w = w_ref[0].astype(jnp.bfloat16)
        y = jnp.dot(x, w, preferred_element_type=jnp.float32)
        a_ref[...] = (y / (1.0 + jnp.exp(-y))).astype(jnp.bfloat16)


def _ffn1(eid, nt, xp, w1):
    grid_spec = pltpu.PrefetchScalarGridSpec(
        num_scalar_prefetch=2,
        grid=(F // BN1, NT),
        in_specs=[
            pl.BlockSpec((BM, D), lambda n, t, eid, nt: (t, 0)),
            pl.BlockSpec((1, D, BN1), lambda n, t, eid, nt: (eid[t], 0, n)),
        ],
        out_specs=pl.BlockSpec((BM, BN1), lambda n, t, eid, nt: (t, n)),
    )
    return pl.pallas_call(
        _ffn1_body,
        grid_spec=grid_spec,
        out_shape=jax.ShapeDtypeStruct((NP, F), jnp.bfloat16),
    )(eid, nt, xp, w1)


def _ffn2_body(eid_ref, nt_ref, a_ref, w_ref, y_ref):
    t = pl.program_id(1)

    @pl.when(t < nt_ref[0])
    def _():
        a = a_ref[...]
        w = w_ref[0].astype(jnp.bfloat16)
        y_ref[...] = jnp.dot(a, w, preferred_element_type=jnp.float32)


def _ffn2(eid, nt, a, w2):
    grid_spec = pltpu.PrefetchScalarGridSpec(
        num_scalar_prefetch=2,
        grid=(D // BN2, NT),
        in_specs=[
            pl.BlockSpec((BM, F), lambda n, t, eid, nt: (t, 0)),
            pl.BlockSpec((1, F, BN2), lambda n, t, eid, nt: (eid[t], 0, n)),
        ],
        out_specs=pl.BlockSpec((BM, BN2), lambda n, t, eid, nt: (t, n)),
    )
    return pl.pallas_call(
        _ffn2_body,
        grid_spec=grid_spec,
        out_shape=jax.ShapeDtypeStruct((NP, D), jnp.float32),
    )(eid, nt, a, w2)


def _gather(yp, pos_flat):
    mesh = plsc.VectorSubcoreMesh(core_axis_name="core",
                                  subcore_axis_name="subcore")

    @functools.partial(
        pl.kernel,
        out_type=jax.ShapeDtypeStruct((2 * N, D), jnp.float32),
        mesh=mesh)
    def k(y_hbm, p_hbm, o_hbm):
        def body(i_vmem, o_vmem):
            pltpu.sync_copy(y_hbm.at[i_vmem.at[0]], o_vmem)

        pltpu.emit_pipeline(
            body,
            grid=(2 * N // W,),
            in_specs=[pl.BlockSpec((1, W), lambda i: (i, 0))],
            out_specs=[pl.BlockSpec((W, D), lambda i: (i, 0))],
            core_axis_name=("core", "subcore"),
            dimension_semantics=(pltpu.PARALLEL,),
        )(p_hbm, o_hbm)

    return k(yp, pos_flat)


def _combine_body(y0_ref, y1_ref, g0_ref, g1_ref, o_ref):
    o_ref[...] = y0_ref[...] * g0_ref[...] + y1_ref[...] * g1_ref[...]


def _combine(yg, g0, g1):
    nb = N // LW
    return pl.pallas_call(
        _combine_body,
        grid=(nb,),
        in_specs=[
            pl.BlockSpec((LW, D), lambda i: (i, 0)),
            pl.BlockSpec((LW, D), lambda i: (i + nb, 0)),
            pl.BlockSpec((LW, 1), lambda i: (i, 0)),
            pl.BlockSpec((LW, 1), lambda i: (i, 0)),
        ],
        out_specs=pl.BlockSpec((LW, D), lambda i: (i, 0)),
        out_shape=jax.ShapeDtypeStruct((N, D), jnp.float32),
    )(yg, yg, g0, g1)


def kernel(h, gate_weight, w1, w2):
    gwt = jnp.zeros((D, LW), jnp.float32).at[:, :E].set(gate_weight.T)
    pos, gw, eid_col, nt = _gate_route(h, gwt)
    pos_rows = pos.T
    eid = eid_col.reshape(LW)[:NT]
    nt = nt.reshape(1)
    xp = _dispatch(h, pos_rows[0].reshape(N // W, W),
                   pos_rows[1].reshape(N // W, W))
    a = _ffn1(eid, nt, xp, w1)
    yp = _ffn2(eid, nt, a, w2)
    yg = _gather(yp, pos_rows.reshape(2 * N // W, W))
    return _combine(yg, gw[:, 0:1], gw[:, 1:2])


# speedup vs baseline: 1.4696x; 1.2937x over previous
import functools

import jax
import jax.numpy as jnp
from jax import lax
from jax.experimental import pallas as pl
from jax.experimental.pallas import tpu as pltpu
from jax.experimental.pallas import tpu_sc as plsc

E = 16
D = 2048
F = 4096
N = 2048
LW = 128
BM = 128
NT = 48
NP = NT * BM
BN1 = 2048
BN2 = 1024
W = 16


def _gate_route_body(h_ref, gwt_ref, pos_ref, gw_ref, eid_ref, nt_ref,
                     o1_ref, o2_ref, r1_ref, r2_ref):
    f32 = jnp.float32
    h = h_ref[...].astype(jnp.bfloat16)
    logits = jnp.dot(h, gwt_ref[...].astype(jnp.bfloat16),
                     preferred_element_type=f32)
    lane = lax.broadcasted_iota(jnp.int32, (N, LW), 1)
    valid = lane < E
    logits = jnp.where(valid, logits, -1e30)
    m = jnp.max(logits, axis=1, keepdims=True)
    p = jnp.where(valid, jnp.exp(logits - m), 0.0)
    scores = p / jnp.sum(p, axis=1, keepdims=True)
    m1 = jnp.max(scores, axis=1, keepdims=True)
    i1 = jnp.min(jnp.where(scores == m1, lane, E), axis=1, keepdims=True)
    oh1 = lane == i1
    sc2 = jnp.where(oh1, -1.0, scores)
    m2 = jnp.max(sc2, axis=1, keepdims=True)
    i2 = jnp.min(jnp.where(sc2 == m2, lane, E), axis=1, keepdims=True)
    oh2 = lane == i2
    gw_ref[:, 0:1] = m1
    gw_ref[:, 1:2] = m2
    o1_ref[...] = oh1.astype(f32)
    o2_ref[...] = oh2.astype(f32)
    c1 = jnp.sum(o1_ref[...], axis=0, keepdims=True)
    c2 = jnp.sum(o2_ref[...], axis=0, keepdims=True)
    counts = c1 + c2
    tilecnt = jnp.floor((counts + (BM - 1)) * (1.0 / BM))
    row = lax.broadcasted_iota(jnp.int32, (LW, LW), 0)
    col = lax.broadcasted_iota(jnp.int32, (LW, LW), 1)
    upper = (row < col).astype(f32)
    poff_t = jnp.dot(tilecnt, upper, preferred_element_type=f32)
    poff_r = poff_t * BM
    nt_ref[...] = jnp.sum(tilecnt).astype(jnp.int32).reshape(1, 1)
    strict = (col < row).astype(f32)
    nb = N // LW

    def blk(b, carry, o_ref, r_ref):
        x = o_ref[pl.ds(b * LW, LW), :]
        cum = jnp.dot(strict, x, preferred_element_type=f32) + carry
        r_ref[pl.ds(b * LW, LW), :] = jnp.sum(x * cum, axis=1, keepdims=True)
        return carry + jnp.sum(x, axis=0, keepdims=True)

    carry = lax.fori_loop(0, nb, lambda b, c: blk(b, c, o1_ref, r1_ref),
                          jnp.zeros((1, LW), f32))
    lax.fori_loop(0, nb, lambda b, c: blk(b, c, o2_ref, r2_ref), carry)
    base1 = jnp.sum(o1_ref[...] * poff_r, axis=1, keepdims=True)
    base2 = jnp.sum(o2_ref[...] * poff_r, axis=1, keepdims=True)
    pos_ref[:, 0:1] = (base1 + r1_ref[...]).astype(jnp.int32)
    pos_ref[:, 1:2] = (base2 + r2_ref[...]).astype(jnp.int32)
    trow = lax.broadcasted_iota(jnp.int32, (LW, LW), 0).astype(f32)
    started = jnp.where((col < E) & (poff_t <= trow), 1.0, 0.0)
    eid_ref[...] = (jnp.sum(started, axis=1, keepdims=True) - 1.0).astype(jnp.int32)


def _gate_route(h, gwt):
    f32 = jnp.float32
    return pl.pallas_call(
        _gate_route_body,
        out_shape=(
            jax.ShapeDtypeStruct((N, 2), jnp.int32),
            jax.ShapeDtypeStruct((N, 2), f32),
            jax.ShapeDtypeStruct((LW, 1), jnp.int32),
            jax.ShapeDtypeStruct((1, 1), jnp.int32),
        ),
        scratch_shapes=[
            pltpu.VMEM((N, LW), f32), pltpu.VMEM((N, LW), f32),
            pltpu.VMEM((N, 1), f32), pltpu.VMEM((N, 1), f32),
        ],
    )(h, gwt)


def _dispatch(h, p0, p1):
    mesh = plsc.VectorSubcoreMesh(core_axis_name="core",
                                  subcore_axis_name="subcore")

    @functools.partial(
        pl.kernel,
        out_type=jax.ShapeDtypeStruct((NP, D), jnp.float32),
        mesh=mesh)
    def k(h_hbm, p0_hbm, p1_hbm, xp_hbm):
        def body(x_vmem, i0_vmem, i1_vmem):
            pltpu.sync_copy(x_vmem, xp_hbm.at[i0_vmem.at[0]])
            pltpu.sync_copy(x_vmem, xp_hbm.at[i1_vmem.at[0]])

        pltpu.emit_pipeline(
            body,
            grid=(N // W,),
            in_specs=[
                pl.BlockSpec((W, D), lambda i: (i, 0)),
                pl.BlockSpec((1, W), lambda i: (i, 0)),
                pl.BlockSpec((1, W), lambda i: (i, 0)),
            ],
            out_specs=[],
            core_axis_name=("core", "subcore"),
            dimension_semantics=(pltpu.PARALLEL,),
        )(h_hbm, p0_hbm, p1_hbm)

    return k(h, p0, p1)


def _ffn1_body(eid_ref, nt_ref, x_ref, w_ref, a_ref):
    t = pl.program_id(1)

    @pl.when(t < nt_ref[0])
    def _():
        x = x_ref[...].astype(jnp.bfloat16)
        w = w_ref[0].astype(jnp.bfloat16)
        y = jnp.dot(x, w, preferred_element_type=jnp.float32)
        a_ref[...] = (y / (1.0 + jnp.exp(-y))).astype(jnp.bfloat16)


def _ffn1(eid, nt, xp, w1):
    grid_spec = pltpu.PrefetchScalarGridSpec(
        num_scalar_prefetch=2,
        grid=(F // BN1, NT),
        in_specs=[
            pl.BlockSpec((BM, D), lambda n, t, eid, nt: (t, 0)),
            pl.BlockSpec((1, D, BN1), lambda n, t, eid, nt: (eid[t], 0, n)),
        ],
        out_specs=pl.BlockSpec((BM, BN1), lambda n, t, eid, nt: (t, n)),
    )
    return pl.pallas_call(
        _ffn1_body,
        grid_spec=grid_spec,
        out_shape=jax.ShapeDtypeStruct((NP, F), jnp.bfloat16),
    )(eid, nt, xp, w1)


def _ffn2_body(eid_ref, nt_ref, a_ref, w_ref, y_ref):
    t = pl.program_id(1)

    @pl.when(t < nt_ref[0])
    def _():
        a = a_ref[...]
        w = w_ref[0].astype(jnp.bfloat16)
        y_ref[...] = jnp.dot(a, w, preferred_element_type=jnp.float32)


def _ffn2(eid, nt, a, w2):
    grid_spec = pltpu.PrefetchScalarGridSpec(
        num_scalar_prefetch=2,
        grid=(D // BN2, NT),
        in_specs=[
            pl.BlockSpec((BM, F), lambda n, t, eid, nt: (t, 0)),
            pl.BlockSpec((1, F, BN2), lambda n, t, eid, nt: (eid[t], 0, n)),
        ],
        out_specs=pl.BlockSpec((BM, BN2), lambda n, t, eid, nt: (t, n)),
    )
    return pl.pallas_call(
        _ffn2_body,
        grid_spec=grid_spec,
        out_shape=jax.ShapeDtypeStruct((NP, D), jnp.float32),
    )(eid, nt, a, w2)


def _gather(yp, pos_flat):
    mesh = plsc.VectorSubcoreMesh(core_axis_name="core",
                                  subcore_axis_name="subcore")

    @functools.partial(
        pl.kernel,
        out_type=jax.ShapeDtypeStruct((2 * N, D), jnp.float32),
        mesh=mesh)
    def k(y_hbm, p_hbm, o_hbm):
        def body(i_vmem, o_vmem):
            pltpu.sync_copy(y_hbm.at[i_vmem.at[0]], o_vmem)

        pltpu.emit_pipeline(
            body,
            grid=(2 * N // W,),
            in_specs=[pl.BlockSpec((1, W), lambda i: (i, 0))],
            out_specs=[pl.BlockSpec((W, D), lambda i: (i, 0))],
            core_axis_name=("core", "subcore"),
            dimension_semantics=(pltpu.PARALLEL,),
        )(p_hbm, o_hbm)

    return k(yp, pos_flat)


def _combine_body(y0_ref, y1_ref, g0_ref, g1_ref, o_ref):
    o_ref[...] = y0_ref[...] * g0_ref[...] + y1_ref[...] * g1_ref[...]


def _combine(yg, g0, g1):
    nb = N // LW
    return pl.pallas_call(
        _combine_body,
        grid=(nb,),
        in_specs=[
            pl.BlockSpec((LW, D), lambda i: (i, 0)),
            pl.BlockSpec((LW, D), lambda i: (i + nb, 0)),
            pl.BlockSpec((LW, 1), lambda i: (i, 0)),
            pl.BlockSpec((LW, 1), lambda i: (i, 0)),
        ],
        out_specs=pl.BlockSpec((LW, D), lambda i: (i, 0)),
        out_shape=jax.ShapeDtypeStruct((N, D), jnp.float32),
    )(yg, yg, g0, g1)


def kernel(h, gate_weight, w1, w2):
    gwt = jnp.zeros((D, LW), jnp.float32).at[:, :E].set(gate_weight.T)
    pos, gw, eid_col, nt = _gate_route(h, gwt)
    pos_rows = pos.T
    eid = eid_col.reshape(LW)[:NT]
    nt = nt.reshape(1)
    xp = _dispatch(h, pos_rows[0].reshape(N // W, W),
                   pos_rows[1].reshape(N // W, W))
    a = _ffn1(eid, nt, xp, w1)
    yp = _ffn2(eid, nt, a, w2)
    yg = _gather(yp, pos_rows.reshape(2 * N // W, W))
    return _combine(yg, gw[:, 0:1], gw[:, 1:2])


# speedup vs baseline: 1.4725x; 1.0020x over previous
import functools

import jax
import jax.numpy as jnp
from jax import lax
from jax.experimental import pallas as pl
from jax.experimental.pallas import tpu as pltpu
from jax.experimental.pallas import tpu_sc as plsc

E = 16
D = 2048
F = 4096
N = 2048
LW = 128
BM = 128
NT = 48
NP = NT * BM
BN1 = 2048
BN2 = 1024
W = 16


def _gate_route_body(h_ref, gwt_ref, pos_ref, gw_ref, eid_ref, nt_ref,
                     o1_ref, o2_ref):
    f32 = jnp.float32
    h = h_ref[...].astype(jnp.bfloat16)
    logits = jnp.dot(h, gwt_ref[...].astype(jnp.bfloat16),
                     preferred_element_type=f32)
    lane = lax.broadcasted_iota(jnp.int32, (N, LW), 1)
    valid = lane < E
    logits = jnp.where(valid, logits, -1e30)
    m = jnp.max(logits, axis=1, keepdims=True)
    p = jnp.where(valid, jnp.exp(logits - m), 0.0)
    scores = p / jnp.sum(p, axis=1, keepdims=True)
    m1 = jnp.max(scores, axis=1, keepdims=True)
    i1 = jnp.min(jnp.where(scores == m1, lane, E), axis=1, keepdims=True)
    oh1 = lane == i1
    sc2 = jnp.where(oh1, -1.0, scores)
    m2 = jnp.max(sc2, axis=1, keepdims=True)
    i2 = jnp.min(jnp.where(sc2 == m2, lane, E), axis=1, keepdims=True)
    oh2 = lane == i2
    gw_ref[:, 0:1] = m1
    gw_ref[:, 1:2] = m2
    o1_ref[...] = oh1.astype(f32)
    o2_ref[...] = oh2.astype(f32)
    c1 = jnp.sum(o1_ref[...], axis=0, keepdims=True)
    c2 = jnp.sum(o2_ref[...], axis=0, keepdims=True)
    counts = c1 + c2
    tilecnt = jnp.floor((counts + (BM - 1)) * (1.0 / BM))
    row = lax.broadcasted_iota(jnp.int32, (LW, LW), 0)
    col = lax.broadcasted_iota(jnp.int32, (LW, LW), 1)
    upper = (row < col).astype(f32)
    poff_t = jnp.dot(tilecnt, upper, preferred_element_type=f32)
    poff_r = poff_t * BM
    nt_ref[...] = jnp.sum(tilecnt).astype(jnp.int32).reshape(1, 1)
    rowN = lax.broadcasted_iota(jnp.int32, (N, N), 0)
    colN = lax.broadcasted_iota(jnp.int32, (N, N), 1)
    strictN = (colN < rowN).astype(jnp.bfloat16)
    O1 = o1_ref[...].astype(jnp.bfloat16)
    O2 = o2_ref[...].astype(jnp.bfloat16)
    cum1 = jnp.dot(strictN, O1, preferred_element_type=f32)
    cum2 = jnp.dot(strictN, O2, preferred_element_type=f32) + c1
    r1 = jnp.sum(o1_ref[...] * cum1, axis=1, keepdims=True)
    r2 = jnp.sum(o2_ref[...] * cum2, axis=1, keepdims=True)
    base1 = jnp.sum(o1_ref[...] * poff_r, axis=1, keepdims=True)
    base2 = jnp.sum(o2_ref[...] * poff_r, axis=1, keepdims=True)
    pos_ref[:, 0:1] = (base1 + r1).astype(jnp.int32)
    pos_ref[:, 1:2] = (base2 + r2).astype(jnp.int32)
    trow = lax.broadcasted_iota(jnp.int32, (LW, LW), 0).astype(f32)
    started = jnp.where((col < E) & (poff_t <= trow), 1.0, 0.0)
    eid_ref[...] = (jnp.sum(started, axis=1, keepdims=True) - 1.0).astype(jnp.int32)


def _gate_route(h, gwt):
    f32 = jnp.float32
    return pl.pallas_call(
        _gate_route_body,
        out_shape=(
            jax.ShapeDtypeStruct((N, 2), jnp.int32),
            jax.ShapeDtypeStruct((N, 2), f32),
            jax.ShapeDtypeStruct((LW, 1), jnp.int32),
            jax.ShapeDtypeStruct((1, 1), jnp.int32),
        ),
        scratch_shapes=[
            pltpu.VMEM((N, LW), f32), pltpu.VMEM((N, LW), f32),
        ],
    )(h, gwt)


def _dispatch(h, p0, p1):
    mesh = plsc.VectorSubcoreMesh(core_axis_name="core",
                                  subcore_axis_name="subcore")

    @functools.partial(
        pl.kernel,
        out_type=jax.ShapeDtypeStruct((NP, D), jnp.float32),
        mesh=mesh)
    def k(h_hbm, p0_hbm, p1_hbm, xp_hbm):
        def body(x_vmem, i0_vmem, i1_vmem):
            pltpu.sync_copy(x_vmem, xp_hbm.at[i0_vmem.at[0]])
            pltpu.sync_copy(x_vmem, xp_hbm.at[i1_vmem.at[0]])

        pltpu.emit_pipeline(
            body,
            grid=(N // W,),
            in_specs=[
                pl.BlockSpec((W, D), lambda i: (i, 0)),
                pl.BlockSpec((1, W), lambda i: (i, 0)),
                pl.BlockSpec((1, W), lambda i: (i, 0)),
            ],
            out_specs=[],
            core_axis_name=("core", "subcore"),
            dimension_semantics=(pltpu.PARALLEL,),
        )(h_hbm, p0_hbm, p1_hbm)

    return k(h, p0, p1)


def _ffn1_body(eid_ref, nt_ref, x_ref, w_ref, a_ref):
    t = pl.program_id(1)

    @pl.when(t < nt_ref[0])
    def _():
        x = x_ref[...].astype(jnp.bfloat16)
        w = w_ref[0].astype(jnp.bfloat16)
        y = jnp.dot(x, w, preferred_element_type=jnp.float32)
        a_ref[...] = (y / (1.0 + jnp.exp(-y))).astype(jnp.bfloat16)


def _ffn1(eid, nt, xp, w1):
    grid_spec = pltpu.PrefetchScalarGridSpec(
        num_scalar_prefetch=2,
        grid=(F // BN1, NT),
        in_specs=[
            pl.BlockSpec((BM, D), lambda n, t, eid, nt: (t, 0)),
            pl.BlockSpec((1, D, BN1), lambda n, t, eid, nt: (eid[t], 0, n)),
        ],
        out_specs=pl.BlockSpec((BM, BN1), lambda n, t, eid, nt: (t, n)),
    )
    return pl.pallas_call(
        _ffn1_body,
        grid_spec=grid_spec,
        out_shape=jax.ShapeDtypeStruct((NP, F), jnp.bfloat16),
    )(eid, nt, xp, w1)


def _ffn2_body(eid_ref, nt_ref, a_ref, w_ref, y_ref):
    t = pl.program_id(1)

    @pl.when(t < nt_ref[0])
    def _():
        a = a_ref[...]
        w = w_ref[0].astype(jnp.bfloat16)
        y_ref[...] = jnp.dot(a, w, preferred_element_type=jnp.float32)


def _ffn2(eid, nt, a, w2):
    grid_spec = pltpu.PrefetchScalarGridSpec(
        num_scalar_prefetch=2,
        grid=(D // BN2, NT),
        in_specs=[
            pl.BlockSpec((BM, F), lambda n, t, eid, nt: (t, 0)),
            pl.BlockSpec((1, F, BN2), lambda n, t, eid, nt: (eid[t], 0, n)),
        ],
        out_specs=pl.BlockSpec((BM, BN2), lambda n, t, eid, nt: (t, n)),
    )
    return pl.pallas_call(
        _ffn2_body,
        grid_spec=grid_spec,
        out_shape=jax.ShapeDtypeStruct((NP, D), jnp.float32),
    )(eid, nt, a, w2)


def _gather(yp, pos_flat):
    mesh = plsc.VectorSubcoreMesh(core_axis_name="core",
                                  subcore_axis_name="subcore")

    @functools.partial(
        pl.kernel,
        out_type=jax.ShapeDtypeStruct((2 * N, D), jnp.float32),
        mesh=mesh)
    def k(y_hbm, p_hbm, o_hbm):
        def body(i_vmem, o_vmem):
            pltpu.sync_copy(y_hbm.at[i_vmem.at[0]], o_vmem)

        pltpu.emit_pipeline(
            body,
            grid=(2 * N // W,),
            in_specs=[pl.BlockSpec((1, W), lambda i: (i, 0))],
            out_specs=[pl.BlockSpec((W, D), lambda i: (i, 0))],
            core_axis_name=("core", "subcore"),
            dimension_semantics=(pltpu.PARALLEL,),
        )(p_hbm, o_hbm)

    return k(yp, pos_flat)


def _combine_body(y0_ref, y1_ref, g0_ref, g1_ref, o_ref):
    o_ref[...] = y0_ref[...] * g0_ref[...] + y1_ref[...] * g1_ref[...]


def _combine(yg, g0, g1):
    nb = N // LW
    return pl.pallas_call(
        _combine_body,
        grid=(nb,),
        in_specs=[
            pl.BlockSpec((LW, D), lambda i: (i, 0)),
            pl.BlockSpec((LW, D), lambda i: (i + nb, 0)),
            pl.BlockSpec((LW, 1), lambda i: (i, 0)),
            pl.BlockSpec((LW, 1), lambda i: (i, 0)),
        ],
        out_specs=pl.BlockSpec((LW, D), lambda i: (i, 0)),
        out_shape=jax.ShapeDtypeStruct((N, D), jnp.float32),
    )(yg, yg, g0, g1)


def kernel(h, gate_weight, w1, w2):
    gwt = jnp.zeros((D, LW), jnp.float32).at[:, :E].set(gate_weight.T)
    pos, gw, eid_col, nt = _gate_route(h, gwt)
    pos_rows = pos.T
    eid = eid_col.reshape(LW)[:NT]
    nt = nt.reshape(1)
    xp = _dispatch(h, pos_rows[0].reshape(N // W, W),
                   pos_rows[1].reshape(N // W, W))
    a = _ffn1(eid, nt, xp, w1)
    yp = _ffn2(eid, nt, a, w2)
    yg = _gather(yp, pos_rows.reshape(2 * N // W, W))
    return _combine(yg, gw[:, 0:1], gw[:, 1:2])
